# baseline (speedup 1.0000x reference)
import functools

import jax
import jax.numpy as jnp
from jax import lax
from jax.experimental import pallas as pl
from jax.experimental.pallas import tpu as pltpu


def _se_body(x_ref, w_ref, o_ref, *, inv_hw, cr):
    xf = x_ref[...].astype(jnp.float32)
    pooled = jnp.sum(xf, axis=1) * inv_hw
    h = lax.dot_general(
        pooled, w_ref[0:cr, :], (((1,), (1,)), ((), ())),
        preferred_element_type=jnp.float32)
    h = jnp.maximum(h, 0.0)
    s = jax.nn.sigmoid(
        jnp.dot(h, w_ref[cr:2 * cr, :], preferred_element_type=jnp.float32))
    o_ref[...] = (xf * s[:, None, :]).astype(o_ref.dtype)


@jax.jit
def kernel(x_nchw, w1, w2):
    B, C, H, W = x_nchw.shape
    HW = H * W
    Cr = w1.shape[0]
    x = x_nchw.transpose(0, 2, 3, 1).reshape(B, HW, C)
    wcat = jnp.concatenate([w1, w2.T], axis=0)

    TB = 8
    while B % TB:
        TB -= 1

    out = pl.pallas_call(
        functools.partial(_se_body, inv_hw=1.0 / float(HW), cr=Cr),
        out_shape=jax.ShapeDtypeStruct((B, HW, C), x.dtype),
        grid=(B // TB,),
        in_specs=[
            pl.BlockSpec((TB, HW, C), lambda b: (b, 0, 0)),
            pl.BlockSpec(wcat.shape, lambda b: (0, 0)),
        ],
        out_specs=pl.BlockSpec((TB, HW, C), lambda b: (b, 0, 0)),
        compiler_params=pltpu.CompilerParams(
            dimension_semantics=("parallel",),
            vmem_limit_bytes=48 << 20,
        ),
        cost_estimate=pl.CostEstimate(
            flops=int(2 * B * C * HW + 4 * B * C * Cr),
            transcendentals=int(B * C),
            bytes_accessed=int(2 * B * C * HW * x.dtype.itemsize),
        ),
    )(x, wcat)
    return out.reshape(B, H, W, C).transpose(0, 3, 1, 2)


# speedup vs baseline: 1.0746x; 1.0746x over previous
import functools

import jax
import jax.numpy as jnp
from jax import lax
from jax.experimental import pallas as pl
from jax.experimental.pallas import tpu as pltpu


def _se_body(x_ref, w_ref, o_ref, *, inv_hw, cr):
    xf = x_ref[...].astype(jnp.float32)
    pooled = jnp.sum(xf, axis=1) * inv_hw
    h = lax.dot_general(
        pooled, w_ref[0:cr, :], (((1,), (1,)), ((), ())),
        preferred_element_type=jnp.float32)
    h = jnp.maximum(h, 0.0)
    s = jax.nn.sigmoid(
        jnp.dot(h, w_ref[cr:2 * cr, :], preferred_element_type=jnp.float32))
    o_ref[...] = (xf * s[:, None, :]).astype(o_ref.dtype)


@jax.jit
def kernel(x_nchw, w1, w2):
    B, C, H, W = x_nchw.shape
    HW = H * W
    Cr = w1.shape[0]
    x = x_nchw.transpose(0, 2, 3, 1).reshape(B, HW, C)
    wcat = jnp.concatenate([w1, w2.T], axis=0)

    TB = 16
    while B % TB:
        TB -= 1

    out = pl.pallas_call(
        functools.partial(_se_body, inv_hw=1.0 / float(HW), cr=Cr),
        out_shape=jax.ShapeDtypeStruct((B, HW, C), x.dtype),
        grid=(B // TB,),
        in_specs=[
            pl.BlockSpec((TB, HW, C), lambda b: (b, 0, 0)),
            pl.BlockSpec(wcat.shape, lambda b: (0, 0)),
        ],
        out_specs=pl.BlockSpec((TB, HW, C), lambda b: (b, 0, 0)),
        compiler_params=pltpu.CompilerParams(
            dimension_semantics=("parallel",),
            vmem_limit_bytes=48 << 20,
        ),
        cost_estimate=pl.CostEstimate(
            flops=int(2 * B * C * HW + 4 * B * C * Cr),
            transcendentals=int(B * C),
            bytes_accessed=int(2 * B * C * HW * x.dtype.itemsize),
        ),
    )(x, wcat)
    return out.reshape(B, H, W, C).transpose(0, 3, 1, 2)
